# baseline (speedup 1.0000x reference)
import functools
import jax
import jax.numpy as jnp
from jax import lax
from jax.experimental import pallas as pl
from jax.experimental.pallas import tpu as pltpu

N_EMBED = 128
N_HEAD = 8
HEAD_SIZE = 16
NUM_EXPERTS = 8
TOP_K = 2
D_FF = 4 * N_EMBED
B = 512
T = 32
SKIP_PROB_THRESHOLD = 0.5
CAPACITY_FACTOR = 1.0

N_TOK = B * T

NB = 64
N_BLK1 = B // NB


def _ln(x, w, b):
    m = jnp.mean(x, axis=-1, keepdims=True)
    v = jnp.mean((x - m) ** 2, axis=-1, keepdims=True)
    return (x - m) / jnp.sqrt(v + 1e-5) * w + b


def _attn_kernel(x_ref, wq_ref, wk_ref, wv_ref, proj_ref, pb_ref,
                 ln1w_ref, ln1b_ref, ln2w_ref, ln2b_ref,
                 rw_ref, rb_ref, nw_ref, nb_ref, sw_ref, sb_ref,
                 noise_ref,
                 x1_ref, flat_ref, w8_ref, ns_ref):
    x = x_ref[...]
    d = N_EMBED
    y = _ln(x, ln1w_ref[...], ln1b_ref[...])
    q = jnp.dot(y, wq_ref[...], preferred_element_type=jnp.float32)
    k = jnp.dot(y, wk_ref[...], preferred_element_type=jnp.float32)
    v = jnp.dot(y, wv_ref[...], preferred_element_type=jnp.float32)
    q3 = q.reshape(NB, T, d)
    k3 = k.reshape(NB, T, d)
    v3 = v.reshape(NB, T, d)
    scale = d ** -0.5
    ti = lax.broadcasted_iota(jnp.int32, (1, T, T), 1)
    tj = lax.broadcasted_iota(jnp.int32, (1, T, T), 2)
    causal = ti >= tj
    outs = []
    for h in range(N_HEAD):
        sl = slice(h * HEAD_SIZE, (h + 1) * HEAD_SIZE)
        qh = q3[:, :, sl]
        kh = k3[:, :, sl]
        vh = v3[:, :, sl]
        s = lax.dot_general(qh, kh, (((2,), (2,)), ((0,), (0,))),
                            preferred_element_type=jnp.float32) * scale
        s = jnp.where(causal, s, -jnp.inf)
        m = jnp.max(s, axis=-1, keepdims=True)
        e = jnp.exp(s - m)
        p = e / jnp.sum(e, axis=-1, keepdims=True)
        ah = lax.dot_general(p, vh, (((2,), (1,)), ((0,), (0,))),
                             preferred_element_type=jnp.float32)
        outs.append(ah)
    att = jnp.concatenate(outs, axis=-1).reshape(NB * T, d)
    x1 = x + jnp.dot(att, proj_ref[...], preferred_element_type=jnp.float32) + pb_ref[...]
    x1_ref[...] = x1
    y2 = _ln(x1, ln2w_ref[...], ln2b_ref[...])
    flat_ref[...] = y2
    logits = jnp.dot(y2, rw_ref[...], preferred_element_type=jnp.float32) + rb_ref[...]
    nlog = jnp.dot(y2, nw_ref[...], preferred_element_type=jnp.float32) + nb_ref[...]
    sp = jnp.maximum(nlog, 0.0) + jnp.log1p(jnp.exp(-jnp.abs(nlog)))
    noisy = logits + noise_ref[...] * sp
    ecol = lax.broadcasted_iota(jnp.int32, noisy.shape, 1)
    m1 = jnp.max(noisy, axis=-1, keepdims=True)
    i1 = jnp.min(jnp.where(noisy == m1, ecol, NUM_EXPERTS), axis=-1, keepdims=True)
    masked = jnp.where(ecol == i1, -jnp.inf, noisy)
    m2 = jnp.max(masked, axis=-1, keepdims=True)
    i2 = jnp.min(jnp.where(masked == m2, ecol, NUM_EXPERTS), axis=-1, keepdims=True)
    z1 = jnp.exp(m1 - m1)
    z2 = jnp.exp(m2 - m1)
    den = z1 + z2
    g1 = z1 / den
    g2 = z2 / den
    sel1 = ecol == i1
    sel2 = ecol == i2
    w8 = jnp.where(sel1, g1, jnp.where(sel2, g2, -1.0))
    w8_ref[...] = w8
    slog = jnp.dot(y2, sw_ref[...], preferred_element_type=jnp.float32) + sb_ref[...]
    ns_ref[...] = jnp.where(slog <= 0.0, 1.0, 0.0)


def _pc1(xf, Wq2, Wk2, Wv2, proj_w, proj_b, ln1_w, ln1_b, ln2_w, ln2_b,
         router_w, router_b, noise_w, noise_b, skip_w, skip_b, noise_c):
    blk = NB * T
    full = lambda shape: pl.BlockSpec(shape, lambda i: tuple(0 for _ in shape))
    grid_spec = pl.GridSpec(
        grid=(N_BLK1,),
        in_specs=[
            pl.BlockSpec((blk, N_EMBED), lambda i: (i, 0)),
            full((N_EMBED, N_EMBED)), full((N_EMBED, N_EMBED)), full((N_EMBED, N_EMBED)),
            full((N_EMBED, N_EMBED)), full((1, N_EMBED)),
            full((1, N_EMBED)), full((1, N_EMBED)), full((1, N_EMBED)), full((1, N_EMBED)),
            full((N_EMBED, NUM_EXPERTS)), full((1, NUM_EXPERTS)),
            full((N_EMBED, NUM_EXPERTS)), full((1, NUM_EXPERTS)),
            full((N_EMBED, 1)), full((1, 1)),
            pl.BlockSpec((blk, NUM_EXPERTS), lambda i: (i, 0)),
        ],
        out_specs=[
            pl.BlockSpec((blk, N_EMBED), lambda i: (i, 0)),
            pl.BlockSpec((blk, N_EMBED), lambda i: (i, 0)),
            pl.BlockSpec((blk, NUM_EXPERTS), lambda i: (i, 0)),
            pl.BlockSpec((blk, 1), lambda i: (i, 0)),
        ],
    )
    out_shapes = [
        jax.ShapeDtypeStruct((N_TOK, N_EMBED), jnp.float32),
        jax.ShapeDtypeStruct((N_TOK, N_EMBED), jnp.float32),
        jax.ShapeDtypeStruct((N_TOK, NUM_EXPERTS), jnp.float32),
        jax.ShapeDtypeStruct((N_TOK, 1), jnp.float32),
    ]
    return pl.pallas_call(_attn_kernel, grid_spec=grid_spec, out_shape=out_shapes)(
        xf, Wq2, Wk2, Wv2, proj_w, proj_b.reshape(1, -1),
        ln1_w.reshape(1, -1), ln1_b.reshape(1, -1), ln2_w.reshape(1, -1), ln2_b.reshape(1, -1),
        router_w, router_b.reshape(1, -1), noise_w, noise_b.reshape(1, -1),
        skip_w, skip_b.reshape(1, 1), noise_c)


def _sum_kernel(ns_ref, out_ref):
    out_ref[...] = jnp.sum(ns_ref[...], keepdims=True).reshape(1, 1)


def _pc2a(ns):
    return pl.pallas_call(
        _sum_kernel,
        out_shape=jax.ShapeDtypeStruct((1, 1), jnp.float32),
    )(ns)


TBLK2 = 512
N_BLK2 = N_TOK // TBLK2


def _rank_kernel(ntok_ref, w8_ref, ns_ref, wm_ref, run_ref):
    i = pl.program_id(0)

    @pl.when(i == 0)
    def _():
        run_ref[...] = jnp.zeros_like(run_ref)

    cap = jnp.floor(ntok_ref[0, 0] * (TOP_K / NUM_EXPERTS) * CAPACITY_FACTOR)
    w8 = w8_ref[...]
    ns = ns_ref[...]
    em = jnp.where((w8 > -0.5) & (ns > 0.5), 1.0, 0.0)
    ri = lax.broadcasted_iota(jnp.int32, (TBLK2, TBLK2), 0)
    ci = lax.broadcasted_iota(jnp.int32, (TBLK2, TBLK2), 1)
    tri = jnp.where(ri >= ci, 1.0, 0.0)
    incl = jnp.dot(tri, em, preferred_element_type=jnp.float32)
    rank = incl - 1.0 + run_ref[...]
    allowed = (em > 0.5) & (rank < cap)
    wm_ref[...] = jnp.where(allowed, jnp.maximum(w8, 0.0), 0.0)
    run_ref[...] = run_ref[...] + incl[TBLK2 - 1:TBLK2, :]


def _pc2b(ntok, w8, ns):
    return pl.pallas_call(
        _rank_kernel,
        grid=(N_BLK2,),
        in_specs=[
            pl.BlockSpec((1, 1), lambda i: (0, 0)),
            pl.BlockSpec((TBLK2, NUM_EXPERTS), lambda i: (i, 0)),
            pl.BlockSpec((TBLK2, 1), lambda i: (i, 0)),
        ],
        out_specs=pl.BlockSpec((TBLK2, NUM_EXPERTS), lambda i: (i, 0)),
        out_shape=jax.ShapeDtypeStruct((N_TOK, NUM_EXPERTS), jnp.float32),
        scratch_shapes=[pltpu.VMEM((1, NUM_EXPERTS), jnp.float32)],
    )(ntok, w8, ns)


TBLK3 = 2048
N_BLK3 = N_TOK // TBLK3


def _ffn_kernel(flat_ref, wm_ref, ew1_ref, eb1_ref, ew2_ref, eb2_ref,
                upd_ref, acc_ref):
    e = pl.program_id(1)

    flat = flat_ref[...]
    h = jnp.maximum(jnp.dot(flat, ew1_ref[0], preferred_element_type=jnp.float32)
                    + eb1_ref[0], 0.0)
    o = jnp.dot(h, ew2_ref[0], preferred_element_type=jnp.float32) + eb2_ref[0]
    eh = lax.broadcasted_iota(jnp.int32, (NUM_EXPERTS, 1), 0)
    onehot = jnp.where(eh == e, 1.0, 0.0)
    w = jnp.dot(wm_ref[...], onehot, preferred_element_type=jnp.float32)
    contrib = w * o

    @pl.when(e == 0)
    def _():
        acc_ref[...] = contrib

    @pl.when(e > 0)
    def _():
        acc_ref[...] = acc_ref[...] + contrib

    @pl.when(e == NUM_EXPERTS - 1)
    def _():
        upd_ref[...] = acc_ref[...]


def _pc3(flat, wm, ew1, eb1, ew2, eb2):
    return pl.pallas_call(
        _ffn_kernel,
        grid=(N_BLK3, NUM_EXPERTS),
        in_specs=[
            pl.BlockSpec((TBLK3, N_EMBED), lambda t, e: (t, 0)),
            pl.BlockSpec((TBLK3, NUM_EXPERTS), lambda t, e: (t, 0)),
            pl.BlockSpec((1, N_EMBED, D_FF), lambda t, e: (e, 0, 0)),
            pl.BlockSpec((1, 1, D_FF), lambda t, e: (e, 0, 0)),
            pl.BlockSpec((1, D_FF, N_EMBED), lambda t, e: (e, 0, 0)),
            pl.BlockSpec((1, 1, N_EMBED), lambda t, e: (e, 0, 0)),
        ],
        out_specs=pl.BlockSpec((TBLK3, N_EMBED), lambda t, e: (t, 0)),
        out_shape=jax.ShapeDtypeStruct((N_TOK, N_EMBED), jnp.float32),
        scratch_shapes=[pltpu.VMEM((TBLK3, N_EMBED), jnp.float32)],
    )(flat, wm, ew1, eb1.reshape(NUM_EXPERTS, 1, D_FF), ew2,
      eb2.reshape(NUM_EXPERTS, 1, N_EMBED))


def _combine_kernel(x1_ref, flat_ref, upd_ref, ns_ref, out_ref):
    ns = ns_ref[...]
    out_ref[...] = x1_ref[...] + jnp.where(ns > 0.5, upd_ref[...], flat_ref[...])


def _pc4(x1, flat, upd, ns):
    grid_spec = pl.GridSpec(
        grid=(N_BLK3,),
        in_specs=[
            pl.BlockSpec((TBLK3, N_EMBED), lambda t: (t, 0)),
            pl.BlockSpec((TBLK3, N_EMBED), lambda t: (t, 0)),
            pl.BlockSpec((TBLK3, N_EMBED), lambda t: (t, 0)),
            pl.BlockSpec((TBLK3, 1), lambda t: (t, 0)),
        ],
        out_specs=pl.BlockSpec((TBLK3, N_EMBED), lambda t: (t, 0)),
    )
    return pl.pallas_call(
        _combine_kernel, grid_spec=grid_spec,
        out_shape=jax.ShapeDtypeStruct((N_TOK, N_EMBED), jnp.float32),
    )(x1, flat, upd, ns)


def kernel(x, Wq, Wk, Wv, proj_w, proj_b, ln1_w, ln1_b, ln2_w, ln2_b,
           router_w, router_b, noise_w, noise_b, skip_w, skip_b,
           ew1, eb1, ew2, eb2):
    xf = x.reshape(N_TOK, N_EMBED)
    Wq2 = Wq.transpose(1, 0, 2).reshape(N_EMBED, N_EMBED)
    Wk2 = Wk.transpose(1, 0, 2).reshape(N_EMBED, N_EMBED)
    Wv2 = Wv.transpose(1, 0, 2).reshape(N_EMBED, N_EMBED)
    noise_c = jax.random.normal(jax.random.key(42), (N_TOK, NUM_EXPERTS),
                                dtype=jnp.float32)
    x1, flat, w8, ns = _pc1(xf, Wq2, Wk2, Wv2, proj_w, proj_b, ln1_w, ln1_b,
                            ln2_w, ln2_b, router_w, router_b, noise_w, noise_b,
                            skip_w, skip_b, noise_c)
    ntok = _pc2a(ns)
    wm = _pc2b(ntok, w8, ns)
    upd = _pc3(flat, wm, ew1, eb1, ew2, eb2)
    out = _pc4(x1, flat, upd, ns)
    return out.reshape(B, T, N_EMBED)
